# baseline (speedup 1.0000x reference)
import functools

import jax
import jax.numpy as jnp
from jax import lax
from jax.experimental import pallas as pl
from jax.experimental.pallas import tpu as pltpu
from jax.experimental.pallas import tpu_sc as plsc

B = 16384
K = 64
NC = 2
NS = 16
NW = NC * NS
BPW = B // NW
CH = BPW // 128
L = 16
KC = K // L

_mesh = plsc.VectorSubcoreMesh(core_axis_name="c", subcore_axis_name="s",
                               num_cores=NC, num_subcores=NS)


@functools.partial(
    pl.kernel,
    out_type=jax.ShapeDtypeStruct((B,), jnp.float32),
    mesh=_mesh,
    compiler_params=pltpu.CompilerParams(needs_layout_passes=False,
                                         use_tc_tiling_on_sc=False),
    scratch_types=[
        pltpu.VMEM((3, CH, 128), jnp.int32),
        pltpu.VMEM((128, K), jnp.int32),
        pltpu.VMEM((128, K), jnp.int32),
        pltpu.VMEM((128, K), jnp.int32),
        pltpu.VMEM((128, K), jnp.int32),
        pltpu.VMEM((128, K), jnp.int32),
        pltpu.VMEM((128, K), jnp.int32),
        pltpu.VMEM((L * BPW,), jnp.float32),
        pltpu.VMEM((BPW,), jnp.float32),
        pltpu.SemaphoreType.DMA((CH,)),
    ],
)
def _parafac_sc(idx_hbm, tab_hbm, out_hbm,
                idx_v, r0a, r0b, r1a, r1b, r2a, r2b, st, outv, sem):
    wid = lax.axis_index("s") * NC + lax.axis_index("c")

    pltpu.sync_copy(idx_hbm.at[wid], idx_v)

    bufs = ((r0a, r1a, r2a), (r0b, r1b, r2b))
    tabs = (tab_hbm, tab_hbm, tab_hbm)

    def fire(j):
        dst = bufs[j % 2]
        return [
            pltpu.async_copy(tabs[t].at[idx_v.at[jnp.int32(t), jnp.int32(j)]],
                             dst[t], sem.at[jnp.int32(j)])
            for t in range(3)
        ]

    lane_stride = lax.iota(jnp.int32, L) * BPW

    def compute(j):
        d0, d1, d2 = bufs[j % 2]

        def row_body(b, carry):
            acc = None
            for c in range(KC):
                g0 = plsc.bitcast(d0[b, pl.ds(c * L, L)], jnp.float32)
                g1 = plsc.bitcast(d1[b, pl.ds(c * L, L)], jnp.float32)
                g2 = plsc.bitcast(d2[b, pl.ds(c * L, L)], jnp.float32)
                p = g0 * g1 * g2
                acc = p if acc is None else acc + p
            plsc.store_scatter(st, [lane_stride + (b + j * 128)], acc)
            return carry

        lax.fori_loop(jnp.int32(0), jnp.int32(128), row_body, jnp.int32(0))

    cps = {0: fire(0), 1: fire(1)}
    for j in range(CH):
        for cp in cps[j]:
            cp.wait()
        compute(j)
        if j + 2 < CH:
            cps[j + 2] = fire(j + 2)

    def red_body(g, carry):
        b0 = g * L
        acc = st[pl.ds(b0, L)]
        for lane in range(1, L):
            acc = acc + st[pl.ds(lane * BPW + b0, L)]
        outv[pl.ds(b0, L)] = acc
        return carry

    lax.fori_loop(jnp.int32(0), jnp.int32(BPW // L), red_body, jnp.int32(0))

    pltpu.sync_copy(outv, out_hbm.at[pl.ds(wid * BPW, BPW)])


def kernel(indices, f0, f1, f2):
    out_dtype = f0.dtype
    big = lax.optimization_barrier(jnp.concatenate([f0, f1, f2], axis=0))
    tab = lax.bitcast_convert_type(big.astype(jnp.float32), jnp.int32)
    off = jnp.array([0, f0.shape[0], 2 * f0.shape[0]], jnp.int32)
    idx = (indices.astype(jnp.int32) + off[:, None]) \
        .reshape(3, NW, CH, 128).transpose(1, 0, 2, 3)
    out = _parafac_sc(idx, tab)
    return out.astype(out_dtype)


# speedup vs baseline: 1.1331x; 1.1331x over previous
import functools

import jax
import jax.numpy as jnp
from jax import lax
from jax.experimental import pallas as pl
from jax.experimental.pallas import tpu as pltpu
from jax.experimental.pallas import tpu_sc as plsc

B = 16384
K = 64
NC = 2
NS = 16
NW = NC * NS
BPW = B // NW
CH = BPW // 128
L = 16
KC = K // L

_mesh = plsc.VectorSubcoreMesh(core_axis_name="c", subcore_axis_name="s",
                               num_cores=NC, num_subcores=NS)


@functools.partial(
    pl.kernel,
    out_type=jax.ShapeDtypeStruct((B,), jnp.float32),
    mesh=_mesh,
    compiler_params=pltpu.CompilerParams(needs_layout_passes=False,
                                         use_tc_tiling_on_sc=False),
    scratch_types=[
        pltpu.VMEM((3, CH, 128), jnp.int32),
        pltpu.VMEM((BPW, K), jnp.float32),
        pltpu.VMEM((BPW, K), jnp.float32),
        pltpu.VMEM((BPW, K), jnp.float32),
        pltpu.VMEM((L * BPW,), jnp.float32),
        pltpu.VMEM((BPW,), jnp.float32),
        pltpu.SemaphoreType.DMA((CH,)),
    ],
)
def _parafac_sc(idx_hbm, f0_hbm, f1_hbm, f2_hbm, out_hbm,
                idx_v, r0, r1, r2, st, outv, sem):
    wid = lax.axis_index("s") * NC + lax.axis_index("c")

    pltpu.sync_copy(idx_hbm.at[wid], idx_v)

    copies = [[] for _ in range(CH)]
    for t, (tab, r) in enumerate(((f0_hbm, r0), (f1_hbm, r1), (f2_hbm, r2))):
        for j in range(CH):
            copies[j].append(
                pltpu.async_copy(tab.at[idx_v.at[jnp.int32(t), jnp.int32(j)]],
                                 r.at[pl.ds(j * 128, 128)], sem.at[jnp.int32(j)]))

    lane_stride = lax.iota(jnp.int32, L) * BPW

    def row_body(b, carry):
        acc = None
        for c in range(KC):
            g0 = r0[b, pl.ds(c * L, L)]
            g1 = r1[b, pl.ds(c * L, L)]
            g2 = r2[b, pl.ds(c * L, L)]
            p = g0 * g1 * g2
            acc = p if acc is None else acc + p
        plsc.store_scatter(st, [lane_stride + b], acc)
        return carry

    for j in range(CH):
        for cp in copies[j]:
            cp.wait()
        lax.fori_loop(jnp.int32(j * 128), jnp.int32((j + 1) * 128),
                      row_body, jnp.int32(0))

    def red_body(g, carry):
        b0 = g * L
        acc = st[pl.ds(b0, L)]
        for lane in range(1, L):
            acc = acc + st[pl.ds(lane * BPW + b0, L)]
        outv[pl.ds(b0, L)] = acc
        return carry

    lax.fori_loop(jnp.int32(0), jnp.int32(BPW // L), red_body, jnp.int32(0))

    pltpu.sync_copy(outv, out_hbm.at[pl.ds(wid * BPW, BPW)])


def kernel(indices, f0, f1, f2):
    out_dtype = f0.dtype
    idx = indices.astype(jnp.int32).reshape(3, NW, CH, 128).transpose(1, 0, 2, 3)
    out = _parafac_sc(idx,
                      f0.astype(jnp.float32),
                      f1.astype(jnp.float32),
                      f2.astype(jnp.float32))
    return out.astype(out_dtype)
